# baseline (speedup 1.0000x reference)
import jax
import jax.numpy as jnp
import numpy as np
from jax import lax
from jax.experimental import pallas as pl

_NNODES = 2000
_K = 30
_DIM = 64
_ALPHA = 3.0
_BR = 200
_NB = _NNODES // _BR


def _threefry2x32(keypair, x1, x2):
    def rotl(x, d):
        return (x << np.uint32(d)) | (x >> np.uint32(32 - d))

    def round4(x1, x2, rots):
        for r in rots:
            x1 = (x1 + x2).astype(np.uint32)
            x2 = rotl(x2, r).astype(np.uint32)
            x2 = (x1 ^ x2).astype(np.uint32)
        return x1, x2

    ks0, ks1 = np.uint32(keypair[0]), np.uint32(keypair[1])
    ks2 = np.uint32(np.uint32(0x1BD11BDA) ^ ks0 ^ ks1)
    ra, rb = (13, 15, 26, 6), (17, 29, 16, 24)
    x1 = (x1 + ks0).astype(np.uint32)
    x2 = (x2 + ks1).astype(np.uint32)
    for i, (rots, ka, kb) in enumerate(
            [(ra, ks1, ks2), (rb, ks2, ks0), (ra, ks0, ks1),
             (rb, ks1, ks2), (ra, ks2, ks0)]):
        x1, x2 = round4(x1, x2, rots)
        x1 = (x1 + ka).astype(np.uint32)
        x2 = (x2 + kb + np.uint32(i + 1)).astype(np.uint32)
    return x1, x2


def _build_noise():
    n = _NNODES * _NNODES
    parts = []
    with np.errstate(over="ignore"):
        for t in range(4):
            ka, kb = _threefry2x32((np.uint32(0), np.uint32(42)),
                                   np.uint32(0), np.uint32(t))
            a, b = _threefry2x32((ka, kb), np.zeros(n, np.uint32),
                                 np.arange(n, dtype=np.uint32))
            bits = (a ^ b).astype(np.uint32)
            fb = ((bits >> np.uint32(9)) | np.uint32(0x3F800000)).astype(np.uint32)
            u = fb.view(np.float32) - np.float32(1.0)
            parts.append((u * np.float32(0.01)).reshape(_NNODES, _NNODES))
    return np.stack(parts)


_NOISE = _build_noise()


def _prep_body(emb_ref, w_ref, b_ref, nv_ref, ham_ref):
    x = lax.dot_general(emb_ref[:], w_ref[:], (((1,), (1,)), ((), ())),
                        preferred_element_type=jnp.float32)
    nv = jnp.tanh(_ALPHA * (x + b_ref[:]))
    nv_ref[:] = nv
    r = nv[:, 0:64]
    i = nv[:, 64:128]
    j = nv[:, 128:192]
    k = nv[:, 192:256]
    ham_ref[:] = jnp.concatenate([
        jnp.concatenate([r, -i, -j, -k], axis=1),
        jnp.concatenate([i, r, -k, j], axis=1),
        jnp.concatenate([j, k, r, -i], axis=1),
        jnp.concatenate([k, -j, i, r], axis=1)], axis=0)


_ROWS = 4 * _BR


def _main_body(nv_ref, ham_ref, nz_ref, o0_ref, o1_ref, o2_ref, o3_ref):
    hb = ham_ref[:].reshape(_ROWS, 4 * _DIM)
    a = lax.dot_general(hb, nv_ref[:], (((1,), (1,)), ((), ())),
                        preferred_element_type=jnp.float32)
    p = jnp.maximum(jnp.tanh(_ALPHA * a), 0.0)
    v = p + nz_ref[:].reshape(_ROWS, _NNODES)

    def count_ge(mid):
        mid_f = lax.bitcast_convert_type(mid, jnp.float32)
        return jnp.sum((v >= mid_f).astype(jnp.int32), axis=1, keepdims=True)

    one_bits = jnp.int32(0x3F800000)
    c_sat = count_ge(jnp.full((_ROWS, 1), one_bits, jnp.int32))
    row_max = jnp.max(v, axis=1, keepdims=True)
    lo0 = jnp.where(c_sat >= _K, one_bits, 0)
    hi0 = lax.bitcast_convert_type(row_max, jnp.int32) + 1

    def bstep(carry):
        lo, hi, cl = carry
        mid = (lo + hi) >> 1
        cnt = count_ge(mid)
        ge = cnt >= _K
        return (jnp.where(ge, mid, lo), jnp.where(ge, hi, mid),
                jnp.where(ge, cnt, cl))

    bcarry = (lo0, hi0, jnp.where(c_sat >= _K, c_sat, _NNODES))
    for _ in range(17):
        bcarry = bstep(bcarry)
    thresh, _, c_ge = lax.cond(
        jnp.max(bcarry[1] - bcarry[0]) <= 1,
        lambda c: c,
        lambda c: lax.fori_loop(0, 13, lambda i, c: bstep(c), c),
        bcarry)

    thresh_f = lax.bitcast_convert_type(thresh, jnp.float32)
    ge = v >= thresh_f
    excess = c_ge - _K
    max_excess = jnp.max(excess)

    outs = (o0_ref, o1_ref, o2_ref, o3_ref)

    def write(res):
        for t in range(4):
            outs[t][...] = res[t * _BR:(t + 1) * _BR]

    @pl.when(max_excess == 0)
    def _():
        write(jnp.where(ge, p, 0.0))

    @pl.when(max_excess == 1)
    def _():
        eq = v == thresh_f
        col = lax.broadcasted_iota(jnp.int32, (_ROWS, _NNODES), 1)
        j_last = jnp.max(jnp.where(eq, col, -1), axis=1, keepdims=True)
        drop = eq & (col == j_last) & (excess == 1)
        write(jnp.where(ge & jnp.logical_not(drop), p, 0.0))

    @pl.when(max_excess > 1)
    def _():
        gt = v > thresh_f
        c_gt = jnp.sum(gt.astype(jnp.int32), axis=1, keepdims=True)
        m = _K - c_gt
        eq = v == thresh_f
        csum = eq.astype(jnp.int32)
        sh = 1
        while sh < _NNODES:
            z = jnp.zeros((_ROWS, sh), jnp.int32)
            csum = csum + jnp.concatenate([z, csum[:, :_NNODES - sh]], axis=1)
            sh *= 2
        mask = gt | (eq & (csum <= m))
        write(jnp.where(mask, p, 0.0))


def kernel(idx, emb, W, b):
    emb = jnp.take(emb, idx, axis=0)
    nv, ham = pl.pallas_call(
        _prep_body,
        out_shape=[
            jax.ShapeDtypeStruct((_NNODES, 4 * _DIM), jnp.float32),
            jax.ShapeDtypeStruct((4 * _NNODES, 4 * _DIM), jnp.float32),
        ],
    )(emb, W, b.reshape(1, -1))
    ham4 = ham.reshape(4, _NNODES, 4 * _DIM)
    part_shape = jax.ShapeDtypeStruct((_NNODES, _NNODES), jnp.float32)
    out = pl.pallas_call(
        _main_body,
        grid=(_NB,),
        in_specs=[
            pl.BlockSpec((_NNODES, 4 * _DIM), lambda rb: (0, 0)),
            pl.BlockSpec((4, _BR, 4 * _DIM), lambda rb: (0, rb, 0)),
            pl.BlockSpec((4, _BR, _NNODES), lambda rb: (0, rb, 0)),
        ],
        out_specs=[pl.BlockSpec((_BR, _NNODES), lambda rb: (rb, 0))] * 4,
        out_shape=[part_shape] * 4,
    )(nv, ham4, _NOISE)
    return tuple(out)


# speedup vs baseline: 1.2652x; 1.2652x over previous
import jax
import jax.numpy as jnp
import numpy as np
from jax import lax
from jax.experimental import pallas as pl

_NNODES = 2000
_K = 30
_DIM = 64
_ALPHA = 3.0
_BR = 200
_NB = _NNODES // _BR


def _threefry2x32(keypair, x1, x2):
    def rotl(x, d):
        return (x << np.uint32(d)) | (x >> np.uint32(32 - d))

    def round4(x1, x2, rots):
        for r in rots:
            x1 = (x1 + x2).astype(np.uint32)
            x2 = rotl(x2, r).astype(np.uint32)
            x2 = (x1 ^ x2).astype(np.uint32)
        return x1, x2

    ks0, ks1 = np.uint32(keypair[0]), np.uint32(keypair[1])
    ks2 = np.uint32(np.uint32(0x1BD11BDA) ^ ks0 ^ ks1)
    ra, rb = (13, 15, 26, 6), (17, 29, 16, 24)
    x1 = (x1 + ks0).astype(np.uint32)
    x2 = (x2 + ks1).astype(np.uint32)
    for i, (rots, ka, kb) in enumerate(
            [(ra, ks1, ks2), (rb, ks2, ks0), (ra, ks0, ks1),
             (rb, ks1, ks2), (ra, ks2, ks0)]):
        x1, x2 = round4(x1, x2, rots)
        x1 = (x1 + ka).astype(np.uint32)
        x2 = (x2 + kb + np.uint32(i + 1)).astype(np.uint32)
    return x1, x2


def _build_noise():
    n = _NNODES * _NNODES
    parts = []
    with np.errstate(over="ignore"):
        for t in range(4):
            ka, kb = _threefry2x32((np.uint32(0), np.uint32(42)),
                                   np.uint32(0), np.uint32(t))
            a, b = _threefry2x32((ka, kb), np.zeros(n, np.uint32),
                                 np.arange(n, dtype=np.uint32))
            bits = (a ^ b).astype(np.uint32)
            fb = ((bits >> np.uint32(9)) | np.uint32(0x3F800000)).astype(np.uint32)
            u = fb.view(np.float32) - np.float32(1.0)
            parts.append((u * np.float32(0.01)).reshape(_NNODES, _NNODES))
    return np.stack(parts)


_NOISE = _build_noise()


def _prep_body(emb_ref, w_ref, b_ref, nv_ref, ham_ref):
    x = lax.dot_general(emb_ref[:], w_ref[:], (((1,), (1,)), ((), ())),
                        preferred_element_type=jnp.float32)
    nv = jnp.tanh(_ALPHA * (x + b_ref[:]))
    nv_ref[:] = nv
    r = nv[:, 0:64]
    i = nv[:, 64:128]
    j = nv[:, 128:192]
    k = nv[:, 192:256]
    ham_ref[:] = jnp.concatenate([
        jnp.concatenate([r, -i, -j, -k], axis=1),
        jnp.concatenate([i, r, -k, j], axis=1),
        jnp.concatenate([j, k, r, -i], axis=1),
        jnp.concatenate([k, -j, i, r], axis=1)], axis=0)


_ROWS = 4 * _BR


def _main_body(nv_ref, ham_ref, nz_ref, o0_ref, o1_ref, o2_ref, o3_ref):
    hb = ham_ref[:].reshape(_ROWS, 4 * _DIM)
    a = lax.dot_general(hb, nv_ref[:], (((1,), (1,)), ((), ())),
                        preferred_element_type=jnp.float32)
    p = jnp.maximum(jnp.tanh(_ALPHA * a), 0.0)
    v = p + nz_ref[:].reshape(_ROWS, _NNODES)

    def count_ge(mid):
        mid_f = lax.bitcast_convert_type(mid, jnp.float32)
        return jnp.sum((v >= mid_f).astype(jnp.int32), axis=1, keepdims=True)

    one_bits = jnp.int32(0x3F800000)
    c_sat = count_ge(jnp.full((_ROWS, 1), one_bits, jnp.int32))
    row_max = jnp.max(v, axis=1, keepdims=True)
    lo0 = jnp.where(c_sat >= _K, one_bits, 0)
    hi0 = lax.bitcast_convert_type(row_max, jnp.int32) + 1

    def bstep(carry):
        lo, hi = carry
        mid = (lo + hi) >> 1
        ge = count_ge(mid) >= _K
        return jnp.where(ge, mid, lo), jnp.where(ge, hi, mid)

    bcarry = (lo0, hi0)
    for _ in range(17):
        bcarry = bstep(bcarry)
    thresh, _ = lax.cond(
        jnp.max(bcarry[1] - bcarry[0]) <= 1,
        lambda c: c,
        lambda c: lax.fori_loop(0, 13, lambda i, c: bstep(c), c),
        bcarry)

    thresh_f = lax.bitcast_convert_type(thresh, jnp.float32)
    ge = v >= thresh_f
    c_ge = count_ge(thresh)
    no_ties = jnp.all(c_ge == _K)

    outs = (o0_ref, o1_ref, o2_ref, o3_ref)

    def write(res):
        for t in range(4):
            outs[t][...] = res[t * _BR:(t + 1) * _BR]

    @pl.when(no_ties)
    def _():
        write(jnp.where(ge, p, 0.0))

    @pl.when(jnp.logical_not(no_ties))
    def _():
        gt = v > thresh_f
        c_gt = jnp.sum(gt.astype(jnp.int32), axis=1, keepdims=True)
        m = _K - c_gt
        eq = v == thresh_f
        csum = eq.astype(jnp.int32)
        sh = 1
        while sh < _NNODES:
            z = jnp.zeros((_ROWS, sh), jnp.int32)
            csum = csum + jnp.concatenate([z, csum[:, :_NNODES - sh]], axis=1)
            sh *= 2
        mask = gt | (eq & (csum <= m))
        write(jnp.where(mask, p, 0.0))


def kernel(idx, emb, W, b):
    emb = jnp.take(emb, idx, axis=0)
    nv, ham = pl.pallas_call(
        _prep_body,
        out_shape=[
            jax.ShapeDtypeStruct((_NNODES, 4 * _DIM), jnp.float32),
            jax.ShapeDtypeStruct((4 * _NNODES, 4 * _DIM), jnp.float32),
        ],
    )(emb, W, b.reshape(1, -1))
    ham4 = ham.reshape(4, _NNODES, 4 * _DIM)
    part_shape = jax.ShapeDtypeStruct((_NNODES, _NNODES), jnp.float32)
    out = pl.pallas_call(
        _main_body,
        grid=(_NB,),
        in_specs=[
            pl.BlockSpec((_NNODES, 4 * _DIM), lambda rb: (0, 0)),
            pl.BlockSpec((4, _BR, 4 * _DIM), lambda rb: (0, rb, 0)),
            pl.BlockSpec((4, _BR, _NNODES), lambda rb: (0, rb, 0)),
        ],
        out_specs=[pl.BlockSpec((_BR, _NNODES), lambda rb: (rb, 0))] * 4,
        out_shape=[part_shape] * 4,
    )(nv, ham4, _NOISE)
    return tuple(out)


# speedup vs baseline: 1.3445x; 1.0627x over previous
import jax
import jax.numpy as jnp
import numpy as np
from jax import lax
from jax.experimental import pallas as pl

_NNODES = 2000
_K = 30
_DIM = 64
_ALPHA = 3.0
_BR = 80
_NB = _NNODES // _BR


def _threefry2x32(keypair, x1, x2):
    def rotl(x, d):
        return (x << np.uint32(d)) | (x >> np.uint32(32 - d))

    def round4(x1, x2, rots):
        for r in rots:
            x1 = (x1 + x2).astype(np.uint32)
            x2 = rotl(x2, r).astype(np.uint32)
            x2 = (x1 ^ x2).astype(np.uint32)
        return x1, x2

    ks0, ks1 = np.uint32(keypair[0]), np.uint32(keypair[1])
    ks2 = np.uint32(np.uint32(0x1BD11BDA) ^ ks0 ^ ks1)
    ra, rb = (13, 15, 26, 6), (17, 29, 16, 24)
    x1 = (x1 + ks0).astype(np.uint32)
    x2 = (x2 + ks1).astype(np.uint32)
    for i, (rots, ka, kb) in enumerate(
            [(ra, ks1, ks2), (rb, ks2, ks0), (ra, ks0, ks1),
             (rb, ks1, ks2), (ra, ks2, ks0)]):
        x1, x2 = round4(x1, x2, rots)
        x1 = (x1 + ka).astype(np.uint32)
        x2 = (x2 + kb + np.uint32(i + 1)).astype(np.uint32)
    return x1, x2


def _build_noise():
    n = _NNODES * _NNODES
    parts = []
    with np.errstate(over="ignore"):
        for t in range(4):
            ka, kb = _threefry2x32((np.uint32(0), np.uint32(42)),
                                   np.uint32(0), np.uint32(t))
            a, b = _threefry2x32((ka, kb), np.zeros(n, np.uint32),
                                 np.arange(n, dtype=np.uint32))
            bits = (a ^ b).astype(np.uint32)
            fb = ((bits >> np.uint32(9)) | np.uint32(0x3F800000)).astype(np.uint32)
            u = fb.view(np.float32) - np.float32(1.0)
            parts.append((u * np.float32(0.01)).reshape(_NNODES, _NNODES))
    return np.stack(parts)


_NOISE = _build_noise()


def _prep_body(emb_ref, w_ref, b_ref, nv_ref, ham_ref):
    x = lax.dot_general(emb_ref[:], w_ref[:], (((1,), (1,)), ((), ())),
                        preferred_element_type=jnp.float32)
    nv = jnp.tanh(_ALPHA * (x + b_ref[:]))
    nv_ref[:] = nv
    r = nv[:, 0:64]
    i = nv[:, 64:128]
    j = nv[:, 128:192]
    k = nv[:, 192:256]
    ham_ref[:] = jnp.concatenate([
        jnp.concatenate([r, -i, -j, -k], axis=1),
        jnp.concatenate([i, r, -k, j], axis=1),
        jnp.concatenate([j, k, r, -i], axis=1),
        jnp.concatenate([k, -j, i, r], axis=1)], axis=0)


_ROWS = 4 * _BR


def _main_body(nv_ref, ham_ref, nz_ref, o0_ref, o1_ref, o2_ref, o3_ref):
    hb = ham_ref[:].reshape(_ROWS, 4 * _DIM)
    a = lax.dot_general(hb, nv_ref[:], (((1,), (1,)), ((), ())),
                        preferred_element_type=jnp.float32)
    p = jnp.maximum(jnp.tanh(_ALPHA * a), 0.0)
    v = p + nz_ref[:].reshape(_ROWS, _NNODES)

    def count_ge(mid):
        mid_f = lax.bitcast_convert_type(mid, jnp.float32)
        return jnp.sum((v >= mid_f).astype(jnp.int32), axis=1, keepdims=True)

    one_bits = jnp.int32(0x3F800000)
    c_sat = count_ge(jnp.full((_ROWS, 1), one_bits, jnp.int32))
    row_max = jnp.max(v, axis=1, keepdims=True)
    lo0 = jnp.where(c_sat >= _K, one_bits, 0)
    hi0 = lax.bitcast_convert_type(row_max, jnp.int32) + 1

    def bstep(carry):
        lo, hi = carry
        mid = (lo + hi) >> 1
        ge = count_ge(mid) >= _K
        return jnp.where(ge, mid, lo), jnp.where(ge, hi, mid)

    bcarry = (lo0, hi0)
    for _ in range(17):
        bcarry = bstep(bcarry)
    thresh, _ = lax.cond(
        jnp.max(bcarry[1] - bcarry[0]) <= 1,
        lambda c: c,
        lambda c: lax.fori_loop(0, 13, lambda i, c: bstep(c), c),
        bcarry)

    thresh_f = lax.bitcast_convert_type(thresh, jnp.float32)
    ge = v >= thresh_f
    c_ge = count_ge(thresh)
    no_ties = jnp.all(c_ge == _K)

    outs = (o0_ref, o1_ref, o2_ref, o3_ref)

    def write(res):
        for t in range(4):
            outs[t][...] = res[t * _BR:(t + 1) * _BR]

    @pl.when(no_ties)
    def _():
        write(jnp.where(ge, p, 0.0))

    @pl.when(jnp.logical_not(no_ties))
    def _():
        gt = v > thresh_f
        c_gt = jnp.sum(gt.astype(jnp.int32), axis=1, keepdims=True)
        m = _K - c_gt
        eq = v == thresh_f
        csum = eq.astype(jnp.int32)
        sh = 1
        while sh < _NNODES:
            z = jnp.zeros((_ROWS, sh), jnp.int32)
            csum = csum + jnp.concatenate([z, csum[:, :_NNODES - sh]], axis=1)
            sh *= 2
        mask = gt | (eq & (csum <= m))
        write(jnp.where(mask, p, 0.0))


def kernel(idx, emb, W, b):
    emb = jnp.take(emb, idx, axis=0)
    nv, ham = pl.pallas_call(
        _prep_body,
        out_shape=[
            jax.ShapeDtypeStruct((_NNODES, 4 * _DIM), jnp.float32),
            jax.ShapeDtypeStruct((4 * _NNODES, 4 * _DIM), jnp.float32),
        ],
    )(emb, W, b.reshape(1, -1))
    ham4 = ham.reshape(4, _NNODES, 4 * _DIM)
    part_shape = jax.ShapeDtypeStruct((_NNODES, _NNODES), jnp.float32)
    out = pl.pallas_call(
        _main_body,
        grid=(_NB,),
        in_specs=[
            pl.BlockSpec((_NNODES, 4 * _DIM), lambda rb: (0, 0)),
            pl.BlockSpec((4, _BR, 4 * _DIM), lambda rb: (0, rb, 0)),
            pl.BlockSpec((4, _BR, _NNODES), lambda rb: (0, rb, 0)),
        ],
        out_specs=[pl.BlockSpec((_BR, _NNODES), lambda rb: (rb, 0))] * 4,
        out_shape=[part_shape] * 4,
    )(nv, ham4, _NOISE)
    return tuple(out)


# speedup vs baseline: 1.4219x; 1.0576x over previous
import jax
import jax.numpy as jnp
import numpy as np
from jax import lax
from jax.experimental import pallas as pl

_NNODES = 2000
_K = 30
_DIM = 64
_ALPHA = 3.0
_BR = 40
_NB = _NNODES // _BR


def _threefry2x32(keypair, x1, x2):
    def rotl(x, d):
        return (x << np.uint32(d)) | (x >> np.uint32(32 - d))

    def round4(x1, x2, rots):
        for r in rots:
            x1 = (x1 + x2).astype(np.uint32)
            x2 = rotl(x2, r).astype(np.uint32)
            x2 = (x1 ^ x2).astype(np.uint32)
        return x1, x2

    ks0, ks1 = np.uint32(keypair[0]), np.uint32(keypair[1])
    ks2 = np.uint32(np.uint32(0x1BD11BDA) ^ ks0 ^ ks1)
    ra, rb = (13, 15, 26, 6), (17, 29, 16, 24)
    x1 = (x1 + ks0).astype(np.uint32)
    x2 = (x2 + ks1).astype(np.uint32)
    for i, (rots, ka, kb) in enumerate(
            [(ra, ks1, ks2), (rb, ks2, ks0), (ra, ks0, ks1),
             (rb, ks1, ks2), (ra, ks2, ks0)]):
        x1, x2 = round4(x1, x2, rots)
        x1 = (x1 + ka).astype(np.uint32)
        x2 = (x2 + kb + np.uint32(i + 1)).astype(np.uint32)
    return x1, x2


def _build_noise():
    n = _NNODES * _NNODES
    parts = []
    with np.errstate(over="ignore"):
        for t in range(4):
            ka, kb = _threefry2x32((np.uint32(0), np.uint32(42)),
                                   np.uint32(0), np.uint32(t))
            a, b = _threefry2x32((ka, kb), np.zeros(n, np.uint32),
                                 np.arange(n, dtype=np.uint32))
            bits = (a ^ b).astype(np.uint32)
            fb = ((bits >> np.uint32(9)) | np.uint32(0x3F800000)).astype(np.uint32)
            u = fb.view(np.float32) - np.float32(1.0)
            parts.append((u * np.float32(0.01)).reshape(_NNODES, _NNODES))
    return np.stack(parts)


_NOISE = _build_noise()


def _prep_body(emb_ref, w_ref, b_ref, nv_ref, ham_ref):
    x = lax.dot_general(emb_ref[:], w_ref[:], (((1,), (1,)), ((), ())),
                        preferred_element_type=jnp.float32)
    nv = jnp.tanh(_ALPHA * (x + b_ref[:]))
    nv_ref[:] = nv
    r = nv[:, 0:64]
    i = nv[:, 64:128]
    j = nv[:, 128:192]
    k = nv[:, 192:256]
    ham_ref[:] = jnp.concatenate([
        jnp.concatenate([r, -i, -j, -k], axis=1),
        jnp.concatenate([i, r, -k, j], axis=1),
        jnp.concatenate([j, k, r, -i], axis=1),
        jnp.concatenate([k, -j, i, r], axis=1)], axis=0)


_ROWS = 4 * _BR


def _main_body(nv_ref, ham_ref, nz_ref, o0_ref, o1_ref, o2_ref, o3_ref):
    hb = ham_ref[:].reshape(_ROWS, 4 * _DIM)
    a = lax.dot_general(hb, nv_ref[:], (((1,), (1,)), ((), ())),
                        preferred_element_type=jnp.float32)
    p = jnp.maximum(jnp.tanh(_ALPHA * a), 0.0)
    v = p + nz_ref[:].reshape(_ROWS, _NNODES)

    def count_ge(mid):
        mid_f = lax.bitcast_convert_type(mid, jnp.float32)
        return jnp.sum((v >= mid_f).astype(jnp.int32), axis=1, keepdims=True)

    one_bits = jnp.int32(0x3F800000)
    c_sat = count_ge(jnp.full((_ROWS, 1), one_bits, jnp.int32))
    row_max = jnp.max(v, axis=1, keepdims=True)
    lo0 = jnp.where(c_sat >= _K, one_bits, 0)
    hi0 = lax.bitcast_convert_type(row_max, jnp.int32) + 1

    def bstep(carry):
        lo, hi = carry
        mid = (lo + hi) >> 1
        ge = count_ge(mid) >= _K
        return jnp.where(ge, mid, lo), jnp.where(ge, hi, mid)

    bcarry = (lo0, hi0)
    for _ in range(17):
        bcarry = bstep(bcarry)
    thresh, _ = lax.cond(
        jnp.max(bcarry[1] - bcarry[0]) <= 1,
        lambda c: c,
        lambda c: lax.fori_loop(0, 13, lambda i, c: bstep(c), c),
        bcarry)

    thresh_f = lax.bitcast_convert_type(thresh, jnp.float32)
    ge = v >= thresh_f
    c_ge = count_ge(thresh)
    no_ties = jnp.all(c_ge == _K)

    outs = (o0_ref, o1_ref, o2_ref, o3_ref)

    def write(res):
        for t in range(4):
            outs[t][...] = res[t * _BR:(t + 1) * _BR]

    @pl.when(no_ties)
    def _():
        write(jnp.where(ge, p, 0.0))

    @pl.when(jnp.logical_not(no_ties))
    def _():
        gt = v > thresh_f
        c_gt = jnp.sum(gt.astype(jnp.int32), axis=1, keepdims=True)
        m = _K - c_gt
        eq = v == thresh_f
        csum = eq.astype(jnp.int32)
        sh = 1
        while sh < _NNODES:
            z = jnp.zeros((_ROWS, sh), jnp.int32)
            csum = csum + jnp.concatenate([z, csum[:, :_NNODES - sh]], axis=1)
            sh *= 2
        mask = gt | (eq & (csum <= m))
        write(jnp.where(mask, p, 0.0))


def kernel(idx, emb, W, b):
    emb = jnp.take(emb, idx, axis=0)
    nv, ham = pl.pallas_call(
        _prep_body,
        out_shape=[
            jax.ShapeDtypeStruct((_NNODES, 4 * _DIM), jnp.float32),
            jax.ShapeDtypeStruct((4 * _NNODES, 4 * _DIM), jnp.float32),
        ],
    )(emb, W, b.reshape(1, -1))
    ham4 = ham.reshape(4, _NNODES, 4 * _DIM)
    part_shape = jax.ShapeDtypeStruct((_NNODES, _NNODES), jnp.float32)
    out = pl.pallas_call(
        _main_body,
        grid=(_NB,),
        in_specs=[
            pl.BlockSpec((_NNODES, 4 * _DIM), lambda rb: (0, 0)),
            pl.BlockSpec((4, _BR, 4 * _DIM), lambda rb: (0, rb, 0)),
            pl.BlockSpec((4, _BR, _NNODES), lambda rb: (0, rb, 0)),
        ],
        out_specs=[pl.BlockSpec((_BR, _NNODES), lambda rb: (rb, 0))] * 4,
        out_shape=[part_shape] * 4,
    )(nv, ham4, _NOISE)
    return tuple(out)
